# Initial kernel scaffold; baseline (speedup 1.0000x reference)
import jax
import jax.numpy as jnp
from jax.experimental import pallas as pl


def kernel(x_lig, x_pro, edge_index_lig, edge_index_pro, edge_index_inter, edge_attr_inter, graph_ids, W_lig, Ws_lig, W_pro, Ws_pro, W_e1, b_e1, W_e2, b_e2, W_f1, b_f1, W_f2, b_f2):
    raise NotImplementedError("write your pallas kernel here")


# speedup vs baseline: 2.9716x; 2.9716x over previous
import functools

import jax
import jax.numpy as jnp
from jax import lax
from jax.experimental import pallas as pl
from jax.experimental.pallas import tpu as pltpu
from jax.experimental.pallas import tpu_sc as plsc

D = 128
NLAYER = 3
NSEG = 64


def _sc_scatter_make(n, e, d):
    C = 80
    n_tiles = 16
    per_tile = e // n_tiles
    n_chunks = per_tile // C
    assert per_tile % C == 0 and per_tile % 8 == 0
    ZR = 80
    rows_t = 640
    assert (n_tiles - 1) * rows_t < n <= n_tiles * rows_t

    mesh = plsc.VectorSubcoreMesh(core_axis_name="c", subcore_axis_name="s")

    @functools.partial(
        pl.kernel,
        out_type=jax.ShapeDtypeStruct((2, n, d), jnp.float32),
        mesh=mesh,
        scratch_types=[
            pltpu.VMEM((C,), jnp.int32),
            pltpu.VMEM((C,), jnp.int32),
            pltpu.VMEM((C, d), jnp.float32),
            pltpu.VMEM((ZR, d), jnp.float32),
            pltpu.VMEM_SHARED((n, d), jnp.float32),
            pltpu.SemaphoreType.DMA,
        ],
    )
    def scat(m_hbm, src_hbm, dst_hbm, out_hbm, idx_s, idx_d, rows, zbuf, acc, sem):
        c = lax.axis_index("c")
        s = lax.axis_index("s")

        def zrow(i, _):
            r = i // (d // 16)
            l = i % (d // 16)
            zbuf[r, pl.ds(l * 16, 16)] = jnp.zeros((16,), jnp.float32)
            return 0

        lax.fori_loop(0, ZR * (d // 16), zrow, 0)
        for k in range(rows_t // ZR):
            r0 = s * rows_t + k * ZR

            @pl.when(r0 < n)
            def _z():
                pltpu.sync_copy(zbuf, acc.at[pl.ds(r0, ZR), :])

        plsc.subcore_barrier()

        def chunk(j, _):
            base = c * e + s * per_tile + j * C
            pltpu.sync_copy(src_hbm.at[pl.ds(base, C)], idx_s)
            pltpu.sync_copy(dst_hbm.at[pl.ds(base, C)], idx_d)
            pltpu.async_copy(m_hbm.at[idx_s], rows, sem).wait()
            pltpu.sync_copy(rows, acc.at[idx_d], add=True)
            return 0

        lax.fori_loop(0, n_chunks, chunk, 0)
        plsc.subcore_barrier()

        for k in range(rows_t // ZR):
            r0 = s * rows_t + k * ZR

            @pl.when(r0 < n)
            def _o():
                pltpu.sync_copy(acc.at[pl.ds(r0, ZR), :],
                                out_hbm.at[c, pl.ds(r0, ZR), :])

    return scat


def _sc_gather_make(nrows, e, d):
    C = 80
    n_tiles = 16
    per_tile = e // n_tiles
    n_chunks = per_tile // C
    assert per_tile % C == 0

    mesh = plsc.VectorSubcoreMesh(core_axis_name="c", subcore_axis_name="s")

    @functools.partial(
        pl.kernel,
        out_type=jax.ShapeDtypeStruct((2, e, d), jnp.float32),
        mesh=mesh,
        scratch_types=[
            pltpu.VMEM((C,), jnp.int32),
            pltpu.VMEM((C, d), jnp.float32),
            pltpu.SemaphoreType.DMA,
        ],
    )
    def gat(table_hbm, idx_hbm, out_hbm, idx_v, rows, sem):
        c = lax.axis_index("c")
        s = lax.axis_index("s")

        def chunk(j, _):
            base = s * per_tile + j * C
            pltpu.sync_copy(idx_hbm.at[pl.ds(c * e + base, C)], idx_v)
            pltpu.async_copy(table_hbm.at[idx_v], rows, sem).wait()
            pltpu.sync_copy(rows, out_hbm.at[c, pl.ds(base, C), :])
            return 0

        lax.fori_loop(0, n_chunks, chunk, 0)

    return gat


def _tc_matmul(xcat, w2):
    n2 = xcat.shape[0]
    BR = 1000
    nb = n2 // BR
    half = nb // 2

    def body(x_ref, w_ref, o_ref):
        o_ref[...] = jnp.dot(x_ref[...], w_ref[0],
                             preferred_element_type=jnp.float32)

    return pl.pallas_call(
        body,
        grid=(nb,),
        in_specs=[
            pl.BlockSpec((BR, D), lambda j: (j, 0)),
            pl.BlockSpec((1, D, D), lambda j: (j // half, 0, 0)),
        ],
        out_specs=pl.BlockSpec((BR, D), lambda j: (j, 0)),
        out_shape=jax.ShapeDtypeStruct((n2, D), jnp.float32),
    )(xcat, w2)


def _tc_fuse(xcat, agg, ws2, wn2):
    n2 = xcat.shape[0]
    BR = 1000
    nb = n2 // BR
    half = nb // 2
    with_next = wn2 is not None

    def body(x_ref, a_ref, ws_ref, *rest):
        if with_next:
            wn_ref, xo_ref, mo_ref = rest
        else:
            (xo_ref,) = rest
        t = jnp.maximum(
            a_ref[...] + jnp.dot(x_ref[...], ws_ref[0],
                                 preferred_element_type=jnp.float32), 0.0)
        xo_ref[...] = t
        if with_next:
            mo_ref[...] = jnp.dot(t, wn_ref[0],
                                  preferred_element_type=jnp.float32)

    in_specs = [
        pl.BlockSpec((BR, D), lambda j: (j, 0)),
        pl.BlockSpec((BR, D), lambda j: (j, 0)),
        pl.BlockSpec((1, D, D), lambda j: (j // half, 0, 0)),
    ]
    args = [xcat, agg, ws2]
    if with_next:
        in_specs.append(pl.BlockSpec((1, D, D), lambda j: (j // half, 0, 0)))
        args.append(wn2)
        out_specs = [pl.BlockSpec((BR, D), lambda j: (j, 0))] * 2
        out_shape = [jax.ShapeDtypeStruct((n2, D), jnp.float32)] * 2
    else:
        out_specs = pl.BlockSpec((BR, D), lambda j: (j, 0))
        out_shape = jax.ShapeDtypeStruct((n2, D), jnp.float32)

    return pl.pallas_call(
        body,
        grid=(nb,),
        in_specs=in_specs,
        out_specs=out_specs,
        out_shape=out_shape,
    )(*args)


def _tc_edge(hs_hd, ea, ids2d, w1a, w1b, w1c, b1, w2, b2, wf1, bf1, wf2, bf2):
    e = ea.shape[0]
    BLK = 1600
    nb = e // BLK
    fh = w1a.shape[1]
    od = w2.shape[1]

    def body(hs_ref, hd_ref, ea_ref, ids_ref, w1a_ref, w1b_ref, w1c_ref,
             b1_ref, w2_ref, b2_ref, wf1_ref, bf1_ref, wf2_ref, bf2_ref,
             out_ref, gs_ref, gm_ref):
        j = pl.program_id(0)

        @pl.when(j == 0)
        def _init():
            gs_ref[...] = jnp.zeros_like(gs_ref)
            gm_ref[...] = jnp.full_like(gm_ref, -jnp.inf)

        ein = (jnp.dot(hs_ref[0], w1a_ref[...], preferred_element_type=jnp.float32)
               + jnp.dot(hd_ref[0], w1b_ref[...], preferred_element_type=jnp.float32)
               + jnp.dot(ea_ref[...], w1c_ref[...], preferred_element_type=jnp.float32)
               + b1_ref[...])
        eact = jnp.maximum(ein, 0.0)
        bond = jnp.maximum(
            jnp.dot(eact, w2_ref[...], preferred_element_type=jnp.float32)
            + b2_ref[...], 0.0)

        ids = ids_ref[...]
        id0 = ids[0, 0]
        id1 = ids[BLK - 1, 0]
        row_iota = lax.broadcasted_iota(jnp.int32, (NSEG, 1), 0)

        def seg_body(g, carry):
            gs, gm = carry
            mask = ids == g
            bsum = jnp.sum(jnp.where(mask, bond, 0.0), axis=0, keepdims=True)
            bmax = jnp.max(jnp.where(mask, bond, -jnp.inf), axis=0,
                           keepdims=True)
            sel = row_iota == g
            gs = jnp.where(sel, gs + bsum, gs)
            gm = jnp.where(sel, jnp.maximum(gm, bmax), gm)
            return gs, gm

        gs, gm = lax.fori_loop(id0, id1 + 1, seg_body,
                               (gs_ref[...], gm_ref[...]))
        gs_ref[...] = gs
        gm_ref[...] = gm

        @pl.when(j == nb - 1)
        def _fin():
            emb = jnp.concatenate([gs, gm], axis=1)
            h = jnp.maximum(
                jnp.dot(emb, wf1_ref[...], preferred_element_type=jnp.float32)
                + bf1_ref[...], 0.0)
            out_ref[...] = (jnp.dot(h, wf2_ref[...],
                                    preferred_element_type=jnp.float32)
                            + bf2_ref[...])

    full = lambda a: pl.BlockSpec(a.shape, lambda j: (0,) * a.ndim)
    in_specs = [
        pl.BlockSpec((1, BLK, D), lambda j: (0, j, 0)),
        pl.BlockSpec((1, BLK, D), lambda j: (1, j, 0)),
        pl.BlockSpec((BLK, ea.shape[1]), lambda j: (j, 0)),
        pl.BlockSpec((BLK, 1), lambda j: (j, 0)),
        full(w1a), full(w1b), full(w1c), full(b1), full(w2), full(b2),
        full(wf1), full(bf1), full(wf2), full(bf2),
    ]

    return pl.pallas_call(
        body,
        grid=(nb,),
        in_specs=in_specs,
        out_specs=pl.BlockSpec((NSEG, 1), lambda j: (0, 0)),
        out_shape=jax.ShapeDtypeStruct((NSEG, 1), jnp.float32),
        scratch_shapes=[
            pltpu.VMEM((NSEG, od), jnp.float32),
            pltpu.VMEM((NSEG, od), jnp.float32),
        ],
    )(hs_hd, hs_hd, ea, ids2d, w1a, w1b, w1c, b1, w2, b2, wf1, bf1, wf2, bf2)


def kernel(x_lig, x_pro, edge_index_lig, edge_index_pro, edge_index_inter,
           edge_attr_inter, graph_ids, W_lig, Ws_lig, W_pro, Ws_pro,
           W_e1, b_e1, W_e2, b_e2, W_f1, b_f1, W_f2, b_f2):
    n = x_lig.shape[0]
    e = edge_index_lig.shape[1]
    ei = edge_index_inter.shape[1]

    xcat = jnp.concatenate([x_lig, x_pro], axis=0)
    src_g = jnp.concatenate([edge_index_lig[0], edge_index_pro[0] + n])
    dst_l = jnp.concatenate([edge_index_lig[1], edge_index_pro[1]])
    Wcat = jnp.stack([W_lig, W_pro], axis=1)
    Wscat = jnp.stack([Ws_lig, Ws_pro], axis=1)

    scat = _sc_scatter_make(n, e, D)
    m = _tc_matmul(xcat, Wcat[0])
    for i in range(NLAYER):
        agg = scat(m, src_g, dst_l).reshape(2 * n, D)
        if i < NLAYER - 1:
            xcat, m = _tc_fuse(xcat, agg, Wscat[i], Wcat[i + 1])
        else:
            xcat = _tc_fuse(xcat, agg, Wscat[i], None)

    gat = _sc_gather_make(2 * n, ei, D)
    hs_hd = gat(xcat, edge_index_inter.reshape(-1))

    w1a = W_e1[:D]
    w1b = W_e1[D:2 * D]
    w1c = W_e1[2 * D:]
    out = _tc_edge(hs_hd, edge_attr_inter, graph_ids.reshape(-1, 1),
                   w1a, w1b, w1c, b_e1.reshape(1, -1),
                   W_e2, b_e2.reshape(1, -1),
                   W_f1, b_f1.reshape(1, -1),
                   W_f2, b_f2.reshape(1, -1))
    return out


# speedup vs baseline: 4.9253x; 1.6575x over previous
import functools

import jax
import jax.numpy as jnp
from jax import lax
from jax.experimental import pallas as pl
from jax.experimental.pallas import tpu as pltpu
from jax.experimental.pallas import tpu_sc as plsc

D = 128
NLAYER = 3
NSEG = 64


def _sc_scatter_make(n, e, d):
    C = 80
    NB = 2
    n_tiles = 16
    per_tile = e // n_tiles
    n_chunks = per_tile // C
    assert per_tile % C == 0 and per_tile % 8 == 0 and n_chunks % NB == 0
    ZR = 80
    rows_t = 640
    assert (n_tiles - 1) * rows_t < n <= n_tiles * rows_t

    mesh = plsc.VectorSubcoreMesh(core_axis_name="c", subcore_axis_name="s")

    @functools.partial(
        pl.kernel,
        out_type=jax.ShapeDtypeStruct((2, n, d), jnp.float32),
        mesh=mesh,
        scratch_types=(
            [pltpu.VMEM((per_tile,), jnp.int32),
             pltpu.VMEM_SHARED((n, d), jnp.float32)]
            + [pltpu.VMEM((C,), jnp.int32)] * NB
            + [pltpu.VMEM((C, d), jnp.float32)] * NB
            + [pltpu.SemaphoreType.DMA] * (3 * NB)
        ),
    )
    def scat(m_hbm, src_hbm, dst_hbm, out_hbm, idx_ss, acc, *bufs):
        idx_d = bufs[:NB]
        rows = bufs[NB:2 * NB]
        sem_i = bufs[2 * NB:3 * NB]
        sem_g = bufs[3 * NB:4 * NB]
        sem_s = bufs[4 * NB:]
        c = lax.axis_index("c")
        s = lax.axis_index("s")

        ebase = c * e + s * per_tile
        hi = pltpu.async_copy(src_hbm.at[pl.ds(ebase, per_tile)], idx_ss,
                              sem_g[0])

        def zrow(i, _):
            r = i // (d // 16)
            l = i % (d // 16)
            rows[0][r, pl.ds(l * 16, 16)] = jnp.zeros((16,), jnp.float32)
            return 0

        lax.fori_loop(0, C * (d // 16), zrow, 0)
        for k in range(rows_t // C):
            r0 = s * rows_t + k * C

            @pl.when(r0 < n)
            def _z():
                pltpu.sync_copy(rows[0], acc.at[pl.ds(r0, C), :])

        hi.wait()
        plsc.subcore_barrier()

        def group(G, _):
            @pl.when(G > 0)
            def _w():
                for b in range(NB):
                    pltpu.make_async_copy(
                        rows[b], acc.at[idx_d[b]], sem_s[b]).wait()

            hi_ = []
            hg = []
            for b in range(NB):
                k = G * NB + b
                hi_.append(pltpu.async_copy(
                    dst_hbm.at[pl.ds(ebase + k * C, C)], idx_d[b], sem_i[b]))
                hg.append(pltpu.async_copy(
                    m_hbm.at[idx_ss.at[pl.ds(k * C, C)]], rows[b], sem_g[b]))
            for b in range(NB):
                hi_[b].wait()
                hg[b].wait()
                pltpu.async_copy(rows[b], acc.at[idx_d[b]], sem_s[b],
                                 add=True)
            return 0

        lax.fori_loop(0, n_chunks // NB, group, 0)
        for b in range(NB):
            pltpu.make_async_copy(rows[b], acc.at[idx_d[b]], sem_s[b]).wait()
        plsc.subcore_barrier()

        for k in range(rows_t // ZR):
            r0 = s * rows_t + k * ZR

            @pl.when(r0 < n)
            def _o():
                pltpu.sync_copy(acc.at[pl.ds(r0, ZR), :],
                                out_hbm.at[c, pl.ds(r0, ZR), :])

    return scat


def _sc_gather_make(nrows, e, d):
    C = 80
    NB = 5
    n_tiles = 16
    per_tile = e // n_tiles
    n_chunks = per_tile // C
    assert per_tile % C == 0 and n_chunks % NB == 0

    mesh = plsc.VectorSubcoreMesh(core_axis_name="c", subcore_axis_name="s")

    @functools.partial(
        pl.kernel,
        out_type=jax.ShapeDtypeStruct((2, e, d), jnp.float32),
        mesh=mesh,
        scratch_types=(
            [pltpu.VMEM((per_tile,), jnp.int32)]
            + [pltpu.VMEM((C, d), jnp.float32)] * NB
            + [pltpu.SemaphoreType.DMA] * (2 * NB)
        ),
    )
    def gat(table_hbm, idx_hbm, out_hbm, idx_stage, *bufs):
        rows = bufs[:NB]
        sem_g = bufs[NB:2 * NB]
        sem_o = bufs[2 * NB:]
        c = lax.axis_index("c")
        s = lax.axis_index("s")

        pltpu.sync_copy(idx_hbm.at[pl.ds(c * e + s * per_tile, per_tile)],
                        idx_stage)

        def group(G, _):
            hs = []
            for b in range(NB):
                k = G * NB + b
                hs.append(pltpu.async_copy(
                    table_hbm.at[idx_stage.at[pl.ds(k * C, C)]], rows[b],
                    sem_g[b]))
            os = []
            for b in range(NB):
                k = G * NB + b
                hs[b].wait()
                os.append(pltpu.async_copy(
                    rows[b], out_hbm.at[c, pl.ds(s * per_tile + k * C, C), :],
                    sem_o[b]))
            for b in range(NB):
                os[b].wait()
            return 0

        lax.fori_loop(0, n_chunks // NB, group, 0)

    return gat


def _tc_matmul(xcat, w2):
    n2 = xcat.shape[0]
    BR = 1000
    nb = n2 // BR
    half = nb // 2

    def body(x_ref, w_ref, o_ref):
        o_ref[...] = jnp.dot(x_ref[...], w_ref[0],
                             preferred_element_type=jnp.float32)

    return pl.pallas_call(
        body,
        grid=(nb,),
        in_specs=[
            pl.BlockSpec((BR, D), lambda j: (j, 0)),
            pl.BlockSpec((1, D, D), lambda j: (j // half, 0, 0)),
        ],
        out_specs=pl.BlockSpec((BR, D), lambda j: (j, 0)),
        out_shape=jax.ShapeDtypeStruct((n2, D), jnp.float32),
    )(xcat, w2)


def _tc_fuse(xcat, agg, ws2, wn2):
    n2 = xcat.shape[0]
    BR = 1000
    nb = n2 // BR
    half = nb // 2
    with_next = wn2 is not None

    def body(x_ref, a_ref, ws_ref, *rest):
        if with_next:
            wn_ref, xo_ref, mo_ref = rest
        else:
            (xo_ref,) = rest
        t = jnp.maximum(
            a_ref[...] + jnp.dot(x_ref[...], ws_ref[0],
                                 preferred_element_type=jnp.float32), 0.0)
        xo_ref[...] = t
        if with_next:
            mo_ref[...] = jnp.dot(t, wn_ref[0],
                                  preferred_element_type=jnp.float32)

    in_specs = [
        pl.BlockSpec((BR, D), lambda j: (j, 0)),
        pl.BlockSpec((BR, D), lambda j: (j, 0)),
        pl.BlockSpec((1, D, D), lambda j: (j // half, 0, 0)),
    ]
    args = [xcat, agg, ws2]
    if with_next:
        in_specs.append(pl.BlockSpec((1, D, D), lambda j: (j // half, 0, 0)))
        args.append(wn2)
        out_specs = [pl.BlockSpec((BR, D), lambda j: (j, 0))] * 2
        out_shape = [jax.ShapeDtypeStruct((n2, D), jnp.float32)] * 2
    else:
        out_specs = pl.BlockSpec((BR, D), lambda j: (j, 0))
        out_shape = jax.ShapeDtypeStruct((n2, D), jnp.float32)

    return pl.pallas_call(
        body,
        grid=(nb,),
        in_specs=in_specs,
        out_specs=out_specs,
        out_shape=out_shape,
    )(*args)


def _tc_edge(hs_hd, ea, ids2d, w1a, w1b, w1c, b1, w2, b2, wf1, bf1, wf2, bf2):
    e = ea.shape[0]
    BLK = 1600
    nb = e // BLK
    fh = w1a.shape[1]
    od = w2.shape[1]

    def body(hs_ref, hd_ref, ea_ref, ids_ref, w1a_ref, w1b_ref, w1c_ref,
             b1_ref, w2_ref, b2_ref, wf1_ref, bf1_ref, wf2_ref, bf2_ref,
             out_ref, gs_ref, gm_ref):
        j = pl.program_id(0)

        @pl.when(j == 0)
        def _init():
            gs_ref[...] = jnp.zeros_like(gs_ref)
            gm_ref[...] = jnp.full_like(gm_ref, -jnp.inf)

        ein = (jnp.dot(hs_ref[0], w1a_ref[...], preferred_element_type=jnp.float32)
               + jnp.dot(hd_ref[0], w1b_ref[...], preferred_element_type=jnp.float32)
               + jnp.dot(ea_ref[...], w1c_ref[...], preferred_element_type=jnp.float32)
               + b1_ref[...])
        eact = jnp.maximum(ein, 0.0)
        bond = jnp.maximum(
            jnp.dot(eact, w2_ref[...], preferred_element_type=jnp.float32)
            + b2_ref[...], 0.0)

        ids = ids_ref[...]
        id0 = ids[0, 0]
        id1 = ids[BLK - 1, 0]
        row_iota = lax.broadcasted_iota(jnp.int32, (NSEG, 1), 0)

        def seg_body(g, carry):
            gs, gm = carry
            mask = ids == g
            bsum = jnp.sum(jnp.where(mask, bond, 0.0), axis=0, keepdims=True)
            bmax = jnp.max(jnp.where(mask, bond, -jnp.inf), axis=0,
                           keepdims=True)
            sel = row_iota == g
            gs = jnp.where(sel, gs + bsum, gs)
            gm = jnp.where(sel, jnp.maximum(gm, bmax), gm)
            return gs, gm

        gs, gm = lax.fori_loop(id0, id1 + 1, seg_body,
                               (gs_ref[...], gm_ref[...]))
        gs_ref[...] = gs
        gm_ref[...] = gm

        @pl.when(j == nb - 1)
        def _fin():
            emb = jnp.concatenate([gs, gm], axis=1)
            h = jnp.maximum(
                jnp.dot(emb, wf1_ref[...], preferred_element_type=jnp.float32)
                + bf1_ref[...], 0.0)
            out_ref[...] = (jnp.dot(h, wf2_ref[...],
                                    preferred_element_type=jnp.float32)
                            + bf2_ref[...])

    full = lambda a: pl.BlockSpec(a.shape, lambda j: (0,) * a.ndim)
    in_specs = [
        pl.BlockSpec((1, BLK, D), lambda j: (0, j, 0)),
        pl.BlockSpec((1, BLK, D), lambda j: (1, j, 0)),
        pl.BlockSpec((BLK, ea.shape[1]), lambda j: (j, 0)),
        pl.BlockSpec((BLK, 1), lambda j: (j, 0)),
        full(w1a), full(w1b), full(w1c), full(b1), full(w2), full(b2),
        full(wf1), full(bf1), full(wf2), full(bf2),
    ]

    return pl.pallas_call(
        body,
        grid=(nb,),
        in_specs=in_specs,
        out_specs=pl.BlockSpec((NSEG, 1), lambda j: (0, 0)),
        out_shape=jax.ShapeDtypeStruct((NSEG, 1), jnp.float32),
        scratch_shapes=[
            pltpu.VMEM((NSEG, od), jnp.float32),
            pltpu.VMEM((NSEG, od), jnp.float32),
        ],
    )(hs_hd, hs_hd, ea, ids2d, w1a, w1b, w1c, b1, w2, b2, wf1, bf1, wf2, bf2)


def kernel(x_lig, x_pro, edge_index_lig, edge_index_pro, edge_index_inter,
           edge_attr_inter, graph_ids, W_lig, Ws_lig, W_pro, Ws_pro,
           W_e1, b_e1, W_e2, b_e2, W_f1, b_f1, W_f2, b_f2):
    n = x_lig.shape[0]
    e = edge_index_lig.shape[1]
    ei = edge_index_inter.shape[1]

    xcat = jnp.concatenate([x_lig, x_pro], axis=0)
    src_g = jnp.concatenate([edge_index_lig[0], edge_index_pro[0] + n])
    dst_l = jnp.concatenate([edge_index_lig[1], edge_index_pro[1]])
    Wcat = jnp.stack([W_lig, W_pro], axis=1)
    Wscat = jnp.stack([Ws_lig, Ws_pro], axis=1)

    scat = _sc_scatter_make(n, e, D)
    m = _tc_matmul(xcat, Wcat[0])
    for i in range(NLAYER):
        agg = scat(m, src_g, dst_l).reshape(2 * n, D)
        if i < NLAYER - 1:
            xcat, m = _tc_fuse(xcat, agg, Wscat[i], Wcat[i + 1])
        else:
            xcat = _tc_fuse(xcat, agg, Wscat[i], None)

    gat = _sc_gather_make(2 * n, ei, D)
    hs_hd = gat(xcat, edge_index_inter.reshape(-1))

    w1a = W_e1[:D]
    w1b = W_e1[D:2 * D]
    w1c = W_e1[2 * D:]
    out = _tc_edge(hs_hd, edge_attr_inter, graph_ids.reshape(-1, 1),
                   w1a, w1b, w1c, b_e1.reshape(1, -1),
                   W_e2, b_e2.reshape(1, -1),
                   W_f1, b_f1.reshape(1, -1),
                   W_f2, b_f2.reshape(1, -1))
    return out


# speedup vs baseline: 5.2702x; 1.0700x over previous
import functools

import jax
import jax.numpy as jnp
from jax import lax
from jax.experimental import pallas as pl
from jax.experimental.pallas import tpu as pltpu
from jax.experimental.pallas import tpu_sc as plsc

D = 128
NLAYER = 3
NSEG = 64


def _sc_scatter_make(n, e, d):
    C = 80
    NB = 4
    n_tiles = 16
    per_tile = e // n_tiles
    n_chunks = per_tile // C
    n_groups = n_chunks // NB
    n_tail = n_chunks - n_groups * NB
    assert per_tile % C == 0 and per_tile % 8 == 0 and n_tail < NB
    ZR = 80
    rows_t = 640
    assert (n_tiles - 1) * rows_t < n <= n_tiles * rows_t

    mesh = plsc.VectorSubcoreMesh(core_axis_name="c", subcore_axis_name="s")

    @functools.partial(
        pl.kernel,
        out_type=jax.ShapeDtypeStruct((2, n, d), jnp.float32),
        mesh=mesh,
        scratch_types=(
            [pltpu.VMEM_SHARED((n, d), jnp.float32)]
            + [pltpu.VMEM((C,), jnp.int32)] * (2 * NB)
            + [pltpu.VMEM((C, d), jnp.float32)] * NB
            + [pltpu.SemaphoreType.DMA] * (3 * NB)
        ),
    )
    def scat(m_hbm, src_hbm, dst_hbm, out_hbm, acc, *bufs):
        idx_s = bufs[:NB]
        idx_d = bufs[NB:2 * NB]
        rows = bufs[2 * NB:3 * NB]
        sem_i = bufs[3 * NB:4 * NB]
        sem_g = bufs[4 * NB:5 * NB]
        sem_s = bufs[5 * NB:]
        c = lax.axis_index("c")
        s = lax.axis_index("s")
        ebase = c * e + s * per_tile

        def zrow(i, _):
            r = i // (d // 16)
            l = i % (d // 16)
            rows[0][r, pl.ds(l * 16, 16)] = jnp.zeros((16,), jnp.float32)
            return 0

        lax.fori_loop(0, C * (d // 16), zrow, 0)
        for k in range(rows_t // C):
            r0 = s * rows_t + k * C

            @pl.when(r0 < n)
            def _z():
                pltpu.sync_copy(rows[0], acc.at[pl.ds(r0, C), :])

        plsc.subcore_barrier()

        def group(G, _):
            his = []
            for b in range(NB):
                k = G * NB + b

                @pl.when(G > 0)
                def _w():
                    pltpu.make_async_copy(
                        rows[b], acc.at[idx_d[b]], sem_s[b]).wait()

                h1 = pltpu.async_copy(
                    src_hbm.at[pl.ds(ebase + k * C, C)], idx_s[b], sem_i[b])
                h2 = pltpu.async_copy(
                    dst_hbm.at[pl.ds(ebase + k * C, C)], idx_d[b], sem_i[b])
                his.append((h1, h2))
            hg = []
            for b in range(NB):
                his[b][0].wait()
                his[b][1].wait()
                hg.append(pltpu.async_copy(
                    m_hbm.at[idx_s[b]], rows[b], sem_g[b]))
            for b in range(NB):
                hg[b].wait()
                pltpu.async_copy(rows[b], acc.at[idx_d[b]], sem_s[b],
                                 add=True)
            return 0

        lax.fori_loop(0, n_groups, group, 0)
        for b in range(n_tail):
            k = n_groups * NB + b
            pltpu.make_async_copy(rows[b], acc.at[idx_d[b]], sem_s[b]).wait()
            h1 = pltpu.async_copy(
                src_hbm.at[pl.ds(ebase + k * C, C)], idx_s[b], sem_i[b])
            h2 = pltpu.async_copy(
                dst_hbm.at[pl.ds(ebase + k * C, C)], idx_d[b], sem_i[b])
            h1.wait()
            h2.wait()
            pltpu.async_copy(m_hbm.at[idx_s[b]], rows[b], sem_g[b]).wait()
            pltpu.async_copy(rows[b], acc.at[idx_d[b]], sem_s[b], add=True)
        for b in range(NB):
            pltpu.make_async_copy(rows[b], acc.at[idx_d[b]], sem_s[b]).wait()
        plsc.subcore_barrier()

        for k in range(rows_t // ZR):
            r0 = s * rows_t + k * ZR

            @pl.when(r0 < n)
            def _o():
                pltpu.sync_copy(acc.at[pl.ds(r0, ZR), :],
                                out_hbm.at[c, pl.ds(r0, ZR), :])

    return scat


def _sc_gather_make(nrows, e, d):
    C = 80
    NB = 5
    n_tiles = 16
    per_tile = e // n_tiles
    n_chunks = per_tile // C
    assert per_tile % C == 0 and n_chunks % NB == 0

    mesh = plsc.VectorSubcoreMesh(core_axis_name="c", subcore_axis_name="s")

    @functools.partial(
        pl.kernel,
        out_type=jax.ShapeDtypeStruct((2, e, d), jnp.float32),
        mesh=mesh,
        scratch_types=(
            [pltpu.VMEM((per_tile,), jnp.int32)]
            + [pltpu.VMEM((C, d), jnp.float32)] * NB
            + [pltpu.SemaphoreType.DMA] * (2 * NB)
        ),
    )
    def gat(table_hbm, idx_hbm, out_hbm, idx_stage, *bufs):
        rows = bufs[:NB]
        sem_g = bufs[NB:2 * NB]
        sem_o = bufs[2 * NB:]
        c = lax.axis_index("c")
        s = lax.axis_index("s")

        pltpu.sync_copy(idx_hbm.at[pl.ds(c * e + s * per_tile, per_tile)],
                        idx_stage)

        def group(G, _):
            hs = []
            for b in range(NB):
                k = G * NB + b

                @pl.when(G > 0)
                def _w():
                    pltpu.make_async_copy(
                        rows[b],
                        out_hbm.at[c, pl.ds(s * per_tile, C), :],
                        sem_o[b]).wait()

                hs.append(pltpu.async_copy(
                    table_hbm.at[idx_stage.at[pl.ds(k * C, C)]], rows[b],
                    sem_g[b]))
            for b in range(NB):
                k = G * NB + b
                hs[b].wait()
                pltpu.async_copy(
                    rows[b], out_hbm.at[c, pl.ds(s * per_tile + k * C, C), :],
                    sem_o[b])
            return 0

        lax.fori_loop(0, n_chunks // NB, group, 0)
        for b in range(NB):
            pltpu.make_async_copy(
                rows[b], out_hbm.at[c, pl.ds(s * per_tile, C), :],
                sem_o[b]).wait()

    return gat


def _tc_matmul(xcat, w2):
    n2 = xcat.shape[0]
    BR = 1000
    nb = n2 // BR
    half = nb // 2

    def body(x_ref, w_ref, o_ref):
        o_ref[...] = jnp.dot(x_ref[...], w_ref[0],
                             preferred_element_type=jnp.float32)

    return pl.pallas_call(
        body,
        grid=(nb,),
        in_specs=[
            pl.BlockSpec((BR, D), lambda j: (j, 0)),
            pl.BlockSpec((1, D, D), lambda j: (j // half, 0, 0)),
        ],
        out_specs=pl.BlockSpec((BR, D), lambda j: (j, 0)),
        out_shape=jax.ShapeDtypeStruct((n2, D), jnp.float32),
    )(xcat, w2)


def _tc_fuse(xcat, agg, ws2, wn2):
    n2 = xcat.shape[0]
    BR = 1000
    nb = n2 // BR
    half = nb // 2
    with_next = wn2 is not None

    def body(x_ref, a_ref, ws_ref, *rest):
        if with_next:
            wn_ref, xo_ref, mo_ref = rest
        else:
            (xo_ref,) = rest
        t = jnp.maximum(
            a_ref[...] + jnp.dot(x_ref[...], ws_ref[0],
                                 preferred_element_type=jnp.float32), 0.0)
        xo_ref[...] = t
        if with_next:
            mo_ref[...] = jnp.dot(t, wn_ref[0],
                                  preferred_element_type=jnp.float32)

    in_specs = [
        pl.BlockSpec((BR, D), lambda j: (j, 0)),
        pl.BlockSpec((BR, D), lambda j: (j, 0)),
        pl.BlockSpec((1, D, D), lambda j: (j // half, 0, 0)),
    ]
    args = [xcat, agg, ws2]
    if with_next:
        in_specs.append(pl.BlockSpec((1, D, D), lambda j: (j // half, 0, 0)))
        args.append(wn2)
        out_specs = [pl.BlockSpec((BR, D), lambda j: (j, 0))] * 2
        out_shape = [jax.ShapeDtypeStruct((n2, D), jnp.float32)] * 2
    else:
        out_specs = pl.BlockSpec((BR, D), lambda j: (j, 0))
        out_shape = jax.ShapeDtypeStruct((n2, D), jnp.float32)

    return pl.pallas_call(
        body,
        grid=(nb,),
        in_specs=in_specs,
        out_specs=out_specs,
        out_shape=out_shape,
    )(*args)


def _tc_edge(hs_hd, ea, ids2d, w1a, w1b, w1c, b1, w2, b2, wf1, bf1, wf2, bf2):
    e = ea.shape[0]
    BLK = 1600
    nb = e // BLK
    fh = w1a.shape[1]
    od = w2.shape[1]

    def body(hs_ref, hd_ref, ea_ref, ids_ref, w1a_ref, w1b_ref, w1c_ref,
             b1_ref, w2_ref, b2_ref, wf1_ref, bf1_ref, wf2_ref, bf2_ref,
             out_ref, gs_ref, gm_ref):
        j = pl.program_id(0)

        @pl.when(j == 0)
        def _init():
            gs_ref[...] = jnp.zeros_like(gs_ref)
            gm_ref[...] = jnp.full_like(gm_ref, -jnp.inf)

        ein = (jnp.dot(hs_ref[0], w1a_ref[...], preferred_element_type=jnp.float32)
               + jnp.dot(hd_ref[0], w1b_ref[...], preferred_element_type=jnp.float32)
               + jnp.dot(ea_ref[...], w1c_ref[...], preferred_element_type=jnp.float32)
               + b1_ref[...])
        eact = jnp.maximum(ein, 0.0)
        bond = jnp.maximum(
            jnp.dot(eact, w2_ref[...], preferred_element_type=jnp.float32)
            + b2_ref[...], 0.0)

        ids = ids_ref[...]
        id0 = ids[0, 0]
        id1 = ids[BLK - 1, 0]
        row_iota = lax.broadcasted_iota(jnp.int32, (NSEG, 1), 0)

        def seg_body(g, carry):
            gs, gm = carry
            mask = ids == g
            bsum = jnp.sum(jnp.where(mask, bond, 0.0), axis=0, keepdims=True)
            bmax = jnp.max(jnp.where(mask, bond, -jnp.inf), axis=0,
                           keepdims=True)
            sel = row_iota == g
            gs = jnp.where(sel, gs + bsum, gs)
            gm = jnp.where(sel, jnp.maximum(gm, bmax), gm)
            return gs, gm

        gs, gm = lax.fori_loop(id0, id1 + 1, seg_body,
                               (gs_ref[...], gm_ref[...]))
        gs_ref[...] = gs
        gm_ref[...] = gm

        @pl.when(j == nb - 1)
        def _fin():
            emb = jnp.concatenate([gs, gm], axis=1)
            h = jnp.maximum(
                jnp.dot(emb, wf1_ref[...], preferred_element_type=jnp.float32)
                + bf1_ref[...], 0.0)
            out_ref[...] = (jnp.dot(h, wf2_ref[...],
                                    preferred_element_type=jnp.float32)
                            + bf2_ref[...])

    full = lambda a: pl.BlockSpec(a.shape, lambda j: (0,) * a.ndim)
    in_specs = [
        pl.BlockSpec((1, BLK, D), lambda j: (0, j, 0)),
        pl.BlockSpec((1, BLK, D), lambda j: (1, j, 0)),
        pl.BlockSpec((BLK, ea.shape[1]), lambda j: (j, 0)),
        pl.BlockSpec((BLK, 1), lambda j: (j, 0)),
        full(w1a), full(w1b), full(w1c), full(b1), full(w2), full(b2),
        full(wf1), full(bf1), full(wf2), full(bf2),
    ]

    return pl.pallas_call(
        body,
        grid=(nb,),
        in_specs=in_specs,
        out_specs=pl.BlockSpec((NSEG, 1), lambda j: (0, 0)),
        out_shape=jax.ShapeDtypeStruct((NSEG, 1), jnp.float32),
        scratch_shapes=[
            pltpu.VMEM((NSEG, od), jnp.float32),
            pltpu.VMEM((NSEG, od), jnp.float32),
        ],
    )(hs_hd, hs_hd, ea, ids2d, w1a, w1b, w1c, b1, w2, b2, wf1, bf1, wf2, bf2)


def kernel(x_lig, x_pro, edge_index_lig, edge_index_pro, edge_index_inter,
           edge_attr_inter, graph_ids, W_lig, Ws_lig, W_pro, Ws_pro,
           W_e1, b_e1, W_e2, b_e2, W_f1, b_f1, W_f2, b_f2):
    n = x_lig.shape[0]
    e = edge_index_lig.shape[1]
    ei = edge_index_inter.shape[1]

    xcat = jnp.concatenate([x_lig, x_pro], axis=0)
    src_g = jnp.concatenate([edge_index_lig[0], edge_index_pro[0] + n])
    dst_l = jnp.concatenate([edge_index_lig[1], edge_index_pro[1]])
    Wcat = jnp.stack([W_lig, W_pro], axis=1)
    Wscat = jnp.stack([Ws_lig, Ws_pro], axis=1)

    scat = _sc_scatter_make(n, e, D)
    m = _tc_matmul(xcat, Wcat[0])
    for i in range(NLAYER):
        agg = scat(m, src_g, dst_l).reshape(2 * n, D)
        if i < NLAYER - 1:
            xcat, m = _tc_fuse(xcat, agg, Wscat[i], Wcat[i + 1])
        else:
            xcat = _tc_fuse(xcat, agg, Wscat[i], None)

    gat = _sc_gather_make(2 * n, ei, D)
    hs_hd = gat(xcat, edge_index_inter.reshape(-1))

    w1a = W_e1[:D]
    w1b = W_e1[D:2 * D]
    w1c = W_e1[2 * D:]
    out = _tc_edge(hs_hd, edge_attr_inter, graph_ids.reshape(-1, 1),
                   w1a, w1b, w1c, b_e1.reshape(1, -1),
                   W_e2, b_e2.reshape(1, -1),
                   W_f1, b_f1.reshape(1, -1),
                   W_f2, b_f2.reshape(1, -1))
    return out
